# baseline (speedup 1.0000x reference)
import functools

import jax
import jax.numpy as jnp
from jax import lax
from jax.experimental import pallas as pl
from jax.experimental.pallas import tpu as pltpu
from jax.experimental.pallas import tpu_sc as plsc

_VOCAB = 1000000
_D = 32
_B = 16384
_K = 20

_NC = 2
_NS = 16
_NW = _NC * _NS
_BPW = _B // _NW
_C = 128
_NCH = _BPW // _C
_G = _C // 16


def _sc_scores(center_r, context_r, neg_r, in_w, out_w):
    mesh = plsc.VectorSubcoreMesh(core_axis_name="c", subcore_axis_name="s")

    @functools.partial(
        pl.kernel,
        out_type=jax.ShapeDtypeStruct((_NW, 1 + _K, _BPW), jnp.float32),
        mesh=mesh,
        compiler_params=pltpu.CompilerParams(
            needs_layout_passes=False, use_tc_tiling_on_sc=False),
        scratch_types=[
            pltpu.VMEM((_NCH, _C), jnp.int32),
            pltpu.VMEM((_NCH, _C), jnp.int32),
            pltpu.VMEM((_NCH, _K, _C), jnp.int32),
            pltpu.VMEM((_C, _D), jnp.float32),
            pltpu.VMEM((_C, _D), jnp.float32),
            pltpu.VMEM((_C * _K, _D), jnp.float32),
            pltpu.VMEM((1 + _K, _BPW), jnp.float32),
            pltpu.SemaphoreType.DMA,
        ],
    )
    def k(center_hbm, context_hbm, neg_hbm, in_hbm, out_hbm, s_hbm,
          cidx_v, xidx_v, nidx_v, v_buf, up_buf, un_buf, s_buf, sem):
        w = lax.axis_index("s") * _NC + lax.axis_index("c")
        pltpu.sync_copy(center_hbm.at[w], cidx_v)
        pltpu.sync_copy(context_hbm.at[w], xidx_v)
        pltpu.sync_copy(neg_hbm.at[w], nidx_v)

        iota = jnp.arange(16, dtype=jnp.int32)

        for c in range(_NCH):
            descs = [
                pltpu.async_copy(in_hbm.at[cidx_v.at[c]], v_buf, sem),
                pltpu.async_copy(out_hbm.at[xidx_v.at[c]], up_buf, sem),
            ]
            for j in range(_K):
                descs.append(pltpu.async_copy(
                    out_hbm.at[nidx_v.at[c, j]],
                    un_buf.at[pl.ds(j * _C, _C)], sem))
            for dsc in descs:
                dsc.wait()

            def group_body(g, _):
                rows = g * 16 + iota
                rows_n = rows * _K

                def d_body(d, accs):
                    dv = jnp.full((16,), 0, jnp.int32) + d
                    vcol = plsc.load_gather(v_buf, [rows, dv])
                    upcol = plsc.load_gather(up_buf, [rows, dv])
                    out = [accs[0] + vcol * upcol]
                    for kk in range(_K):
                        ncol = plsc.load_gather(un_buf, [rows_n + kk, dv])
                        out.append(accs[kk + 1] - vcol * ncol)
                    return tuple(out)

                zero = jnp.zeros((16,), jnp.float32)
                accs = lax.fori_loop(0, _D, d_body, (zero,) * (1 + _K))
                off = c * _C + g * 16
                for r in range(1 + _K):
                    s_buf[r, pl.ds(off, 16)] = accs[r]
                return 0

            lax.fori_loop(0, _G, group_body, 0)

        pltpu.sync_copy(s_buf, s_hbm.at[w])

    return k(center_r, context_r, neg_r, in_w, out_w)


def _tc_loss(scores2d):
    def body(s_ref, o_ref):
        x = s_ref[...]
        total = jnp.sum(jnp.log(jax.nn.sigmoid(x)))
        o_ref[...] = jnp.reshape(-total / _B, (1, 1))

    out = pl.pallas_call(
        body,
        out_shape=jax.ShapeDtypeStruct((1, 1), jnp.float32),
    )(scores2d)
    return out[0, 0]


def kernel(center, context, negative, in_embed_w, out_embed_w):
    center_r = center.astype(jnp.int32).reshape(_NW, _NCH, _C)
    context_r = context.astype(jnp.int32).reshape(_NW, _NCH, _C)
    neg_r = negative.astype(jnp.int32).reshape(_NW, _NCH, _K, _C)
    s = _sc_scores(center_r, context_r, neg_r, in_embed_w, out_embed_w)
    return _tc_loss(s.reshape(_NW * (1 + _K), _BPW))


# speedup vs baseline: 1.0717x; 1.0717x over previous
import functools

import jax
import jax.numpy as jnp
from jax import lax
from jax.experimental import pallas as pl
from jax.experimental.pallas import tpu as pltpu
from jax.experimental.pallas import tpu_sc as plsc

_VOCAB = 1000000
_D = 32
_B = 16384
_K = 20

_NC = 2
_NS = 16
_NW = _NC * _NS
_BPW = _B // _NW
_C = 64
_NCH = _BPW // _C
_NS21 = (1 + _K) * _BPW


def _sc_scores(center, context, neg_t, in_w, out_w):
    mesh = plsc.VectorSubcoreMesh(core_axis_name="c", subcore_axis_name="s")

    @functools.partial(
        pl.kernel,
        out_type=jax.ShapeDtypeStruct((_NW, _NS21), jnp.float32),
        mesh=mesh,
        compiler_params=pltpu.CompilerParams(
            needs_layout_passes=False, use_tc_tiling_on_sc=False),
        scratch_types=[
            pltpu.VMEM((_BPW,), jnp.int32),
            pltpu.VMEM((_BPW,), jnp.int32),
            pltpu.VMEM((_K, _BPW), jnp.int32),
            pltpu.VMEM((_C, _D), jnp.float32),
            pltpu.VMEM((_C, _D), jnp.float32),
            pltpu.VMEM((_C, _D), jnp.float32),
            pltpu.VMEM((_C, _D), jnp.float32),
            pltpu.VMEM((_K * _C, _D), jnp.float32),
            pltpu.VMEM((_K * _C, _D), jnp.float32),
            pltpu.VMEM((_NS21,), jnp.float32),
            pltpu.SemaphoreType.DMA,
            pltpu.SemaphoreType.DMA,
        ],
    )
    def k(center_hbm, context_hbm, neg_hbm, in_hbm, out_hbm, s_hbm,
          cidx_v, xidx_v, nidx_v, v0, v1, up0, up1, un0, un1, s_buf,
          sem0, sem1):
        w = lax.axis_index("s") * _NC + lax.axis_index("c")
        base = w * _BPW
        pltpu.sync_copy(center_hbm.at[pl.ds(base, _BPW)], cidx_v)
        pltpu.sync_copy(context_hbm.at[pl.ds(base, _BPW)], xidx_v)
        pltpu.sync_copy(neg_hbm.at[:, pl.ds(base, _BPW)], nidx_v)

        iota = jnp.arange(16, dtype=jnp.int32)
        mask15 = iota == 15
        bufs = [(v0, up0, un0, sem0), (v1, up1, un1, sem1)]

        def issue(c):
            v_b, up_b, un_b, sem = bufs[c % 2]
            descs = [
                pltpu.async_copy(
                    in_hbm.at[cidx_v.at[pl.ds(c * _C, _C)]], v_b, sem),
                pltpu.async_copy(
                    out_hbm.at[xidx_v.at[pl.ds(c * _C, _C)]], up_b, sem),
            ]
            for kk in range(_K):
                descs.append(pltpu.async_copy(
                    out_hbm.at[nidx_v.at[kk, pl.ds(c * _C, _C)]],
                    un_b.at[pl.ds(kk * _C, _C)], sem))
            return descs

        pending = issue(0)
        for c in range(_NCH):
            nxt = issue(c + 1) if c + 1 < _NCH else []
            for dsc in pending:
                dsc.wait()
            pending = nxt
            v_b, up_b, un_b, _ = bufs[c % 2]

            def b_body(bl, _):
                va = v_b[bl, pl.ds(0, 16)]
                vb = v_b[bl, pl.ds(16, 16)]
                pos = iota * 0 + (c * _C + bl)
                ua = up_b[bl, pl.ds(0, 16)]
                ub = up_b[bl, pl.ds(16, 16)]
                s = plsc.cumsum(va * ua + vb * ub)
                plsc.store_scatter(s_buf, [pos], s, mask=mask15)
                for kk in range(_K):
                    na = un_b[kk * _C + bl, pl.ds(0, 16)]
                    nb = un_b[kk * _C + bl, pl.ds(16, 16)]
                    sn = plsc.cumsum(va * na + vb * nb)
                    plsc.store_scatter(
                        s_buf, [pos + (kk + 1) * _BPW], -sn, mask=mask15)
                return 0

            lax.fori_loop(0, _C, b_body, 0)

        pltpu.sync_copy(s_buf, s_hbm.at[w])

    return k(center, context, neg_t, in_w, out_w)


def _tc_loss(scores2d):
    def body(s_ref, o_ref):
        x = s_ref[...]
        total = jnp.sum(jnp.log(jax.nn.sigmoid(x)))
        o_ref[...] = jnp.reshape(-total / _B, (1, 1))

    out = pl.pallas_call(
        body,
        out_shape=jax.ShapeDtypeStruct((1, 1), jnp.float32),
    )(scores2d)
    return out[0, 0]


def kernel(center, context, negative, in_embed_w, out_embed_w):
    center_i = center.astype(jnp.int32)
    context_i = context.astype(jnp.int32)
    neg_t = negative.astype(jnp.int32).T
    s = _sc_scores(center_i, context_i, neg_t, in_embed_w, out_embed_w)
    return _tc_loss(s.reshape(_NW * (1 + _K), _BPW))
